# baseline (speedup 1.0000x reference)
import functools

import jax
import jax.numpy as jnp
from jax.experimental import pallas as pl
from jax.experimental.pallas import tpu as pltpu

_N = 10000
_D = 256
_NP = 10240
_BR = 512
_E_TOTAL = 170000
_EP = 171008
_BE = 512


def _dense_kernel(x_ref, w_ref, b_ref, o_ref, *, act):
    y = jnp.dot(x_ref[...], w_ref[...], preferred_element_type=jnp.float32)
    y = y + b_ref[...]
    if act:
        y = jnp.maximum(y, 0.0)
    o_ref[...] = y


def _dense(x, w, b=None, act=False, br=_BR):
    m, k = x.shape
    n = w.shape[1]
    if b is None:
        b = jnp.zeros((n,), jnp.float32)
    return pl.pallas_call(
        functools.partial(_dense_kernel, act=act),
        grid=(m // br,),
        in_specs=[pl.BlockSpec((br, k), lambda i: (i, 0)),
                  pl.BlockSpec((k, n), lambda i: (0, 0)),
                  pl.BlockSpec((1, n), lambda i: (0, 0))],
        out_specs=pl.BlockSpec((br, n), lambda i: (i, 0)),
        out_shape=jax.ShapeDtypeStruct((m, n), jnp.float32),
    )(x, w, b.reshape(1, n))


def _pn_edge_kernel(hxg_ref, ps_ref, pd_ref, w1b_ref, b1_ref, w2_ref, b2_ref,
                    o_ref):
    dpos = ps_ref[...] - pd_ref[...]
    h1 = hxg_ref[...] + jnp.dot(dpos, w1b_ref[...],
                                preferred_element_type=jnp.float32)
    h1 = jnp.maximum(h1 + b1_ref[...], 0.0)
    h2 = jnp.dot(h1, w2_ref[...], preferred_element_type=jnp.float32)
    o_ref[...] = jnp.maximum(h2 + b2_ref[...], 0.0)


def _pn_edge(hxg, ps, pd, w1b, b1, w2, b2):
    m = hxg.shape[0]
    return pl.pallas_call(
        _pn_edge_kernel,
        grid=(m // _BE,),
        in_specs=[pl.BlockSpec((_BE, _D), lambda i: (i, 0)),
                  pl.BlockSpec((_BE, 16), lambda i: (i, 0)),
                  pl.BlockSpec((_BE, 16), lambda i: (i, 0)),
                  pl.BlockSpec((16, _D), lambda i: (0, 0)),
                  pl.BlockSpec((1, _D), lambda i: (0, 0)),
                  pl.BlockSpec((_D, _D), lambda i: (0, 0)),
                  pl.BlockSpec((1, _D), lambda i: (0, 0))],
        out_specs=pl.BlockSpec((_BE, _D), lambda i: (i, 0)),
        out_shape=jax.ShapeDtypeStruct((m, _D), jnp.float32),
    )(hxg, ps, pd, w1b, b1.reshape(1, _D), w2, b2.reshape(1, _D))


def _pn_g_kernel(a_ref, w1_ref, b1_ref, w2_ref, b2_ref, o_ref):
    g = jnp.dot(a_ref[...], w1_ref[...],
                preferred_element_type=jnp.float32) + b1_ref[...]
    g = jnp.dot(g, w2_ref[...], preferred_element_type=jnp.float32) + b2_ref[...]
    o_ref[...] = jnp.maximum(g, 0.0)


def _pn_g3_kernel(a_ref, w1_ref, b1_ref, w2_ref, b2_ref, w3_ref, b3_ref,
                  o_ref):
    g = jnp.dot(a_ref[...], w1_ref[...],
                preferred_element_type=jnp.float32) + b1_ref[...]
    g = jnp.dot(g, w2_ref[...], preferred_element_type=jnp.float32) + b2_ref[...]
    g = jnp.maximum(g, 0.0)
    g = jnp.dot(g, w3_ref[...], preferred_element_type=jnp.float32) + b3_ref[...]
    o_ref[...] = jnp.maximum(g, 0.0)


def _pn_g(agg, p, last):
    m = agg.shape[0]
    wspec = pl.BlockSpec((_D, _D), lambda i: (0, 0))
    bspec = pl.BlockSpec((1, _D), lambda i: (0, 0))
    args = [agg, p["gW1"], p["gb1"].reshape(1, _D),
            p["gW2"], p["gb2"].reshape(1, _D)]
    specs = [pl.BlockSpec((_BR, _D), lambda i: (i, 0)), wspec, bspec, wspec,
             bspec]
    body = _pn_g_kernel
    if last:
        args += [p["gW3"], p["gb3"].reshape(1, _D)]
        specs += [wspec, bspec]
        body = _pn_g3_kernel
    return pl.pallas_call(
        body,
        grid=(m // _BR,),
        in_specs=specs,
        out_specs=pl.BlockSpec((_BR, _D), lambda i: (i, 0)),
        out_shape=jax.ShapeDtypeStruct((m, _D), jnp.float32),
    )(*args)


def _ew_kernel(x_ref, b_ref, o_ref, *, act):
    y = x_ref[...] + b_ref[...]
    if act:
        y = jnp.maximum(y, 0.0)
    o_ref[...] = y


def _ew(x, b, act):
    m, n = x.shape
    return pl.pallas_call(
        functools.partial(_ew_kernel, act=act),
        grid=(m // _BR,),
        in_specs=[pl.BlockSpec((_BR, n), lambda i: (i, 0)),
                  pl.BlockSpec((1, n), lambda i: (0, 0))],
        out_specs=pl.BlockSpec((_BR, n), lambda i: (i, 0)),
        out_shape=jax.ShapeDtypeStruct((m, n), jnp.float32),
    )(x, b.reshape(1, n))


def _mlp_kernel(ms_ref, w1_ref, b1_ref, w2_ref, b2_ref, w3_ref, b3_ref,
                o_ref):
    g = jnp.dot(ms_ref[...], w1_ref[...],
                preferred_element_type=jnp.float32) + b1_ref[...]
    g = jnp.dot(g, w2_ref[...], preferred_element_type=jnp.float32) + b2_ref[...]
    g = jnp.maximum(g, 0.0)
    g = jnp.dot(g, w3_ref[...], preferred_element_type=jnp.float32) + b3_ref[...]
    o_ref[...] = g


def _mlp(mode_stats, mp):
    msP = jnp.zeros((8, 8), jnp.float32).at[0, :2].set(mode_stats[0])
    w1P = jnp.zeros((8, 128), jnp.float32).at[:2].set(mp["m1_W"])
    out = pl.pallas_call(
        _mlp_kernel,
        out_shape=jax.ShapeDtypeStruct((8, _D), jnp.float32),
    )(msP, w1P, mp["m1_b"].reshape(1, 128),
      mp["m2_W"], mp["m2_b"].reshape(1, 128),
      mp["m3_W"], mp["m3_b"].reshape(1, _D))
    return out[0:1]


def kernel(x, pos, mode_stats, params, edge_index):
    n = _N
    loops = jnp.arange(n, dtype=edge_index.dtype)
    pad = _EP - _E_TOTAL
    src = jnp.concatenate([edge_index[0], loops,
                           jnp.zeros((pad,), edge_index.dtype)])
    dst = jnp.concatenate([edge_index[1], loops,
                           jnp.full((pad,), n, edge_index.dtype)])
    dst_c = jnp.minimum(dst, n - 1)

    xp = jnp.pad(x, ((0, _NP - n), (0, 0)))
    posall = jnp.pad(pos.reshape(n, 6), ((0, 0), (0, 10)))
    ps = jnp.take(posall, src, axis=0, mode="clip")
    pd = jnp.take(posall, dst_c, axis=0, mode="clip")

    h = xp
    for li, pname in enumerate(("pn1", "pn2", "pn3")):
        p = params[pname]
        w1a = p["lW1"][:_D]
        w1bP = jnp.zeros((16, _D), jnp.float32).at[2 * li:2 * li + 2].set(
            p["lW1"][_D:])
        hx = _dense(h, w1a)
        hxg = jnp.take(hx, src, axis=0, mode="clip")
        h2 = _pn_edge(hxg, ps, pd, w1bP, p["lb1"], p["lW2"], p["lb2"])
        agg = jax.ops.segment_max(h2, dst, num_segments=n + 1)[:n]
        agg = jnp.pad(agg, ((0, _NP - n), (0, 0)))
        h = _pn_g(agg, p, last=(pname == "pn3"))

    for gi, gname in enumerate(("gat1", "gat2")):
        p = params[gname]
        hh = _dense(h, p["W"])
        attW = (jnp.zeros((_D, 128), jnp.float32)
                .at[:, 0].set(p["att_src"]).at[:, 1].set(p["att_dst"]))
        asd = _dense(hh, attW)
        a_s, a_d = asd[:, 0], asd[:, 1]
        e = jax.nn.leaky_relu(jnp.take(a_s, src, mode="clip")
                              + jnp.take(a_d, dst_c, mode="clip"), 0.2)
        mx = jax.ops.segment_max(e, dst, num_segments=n + 1)
        ex = jnp.exp(e - jnp.take(mx, dst, mode="clip"))
        den = jax.ops.segment_sum(ex, dst, num_segments=n + 1)
        alpha = ex / jnp.take(den, dst, mode="clip")
        hhg = jnp.take(hh, src, axis=0, mode="clip")
        out = jax.ops.segment_sum(hhg * alpha[:, None], dst,
                                  num_segments=n + 1)[:n]
        out = jnp.pad(out, ((0, _NP - n), (0, 0)))
        h = _ew(out, p["b"], act=(gi == 0))

    g = _mlp(mode_stats, params["mlp"])
    return (h[:n], g)


# speedup vs baseline: 1.1797x; 1.1797x over previous
import functools

import jax
import jax.numpy as jnp
from jax import lax
from jax.experimental import pallas as pl
from jax.experimental.pallas import tpu as pltpu
from jax.experimental.pallas import tpu_sc as plsc

_N = 10000
_D = 256
_NP = 10240
_BR = 512
_E_TOTAL = 170000
_EP = 171008
_BE = 512


def _dense_kernel(x_ref, w_ref, b_ref, o_ref, *, act):
    y = jnp.dot(x_ref[...], w_ref[...], preferred_element_type=jnp.float32)
    y = y + b_ref[...]
    if act:
        y = jnp.maximum(y, 0.0)
    o_ref[...] = y


def _dense(x, w, b=None, act=False, br=_BR):
    m, k = x.shape
    n = w.shape[1]
    if b is None:
        b = jnp.zeros((n,), jnp.float32)
    return pl.pallas_call(
        functools.partial(_dense_kernel, act=act),
        grid=(m // br,),
        in_specs=[pl.BlockSpec((br, k), lambda i: (i, 0)),
                  pl.BlockSpec((k, n), lambda i: (0, 0)),
                  pl.BlockSpec((1, n), lambda i: (0, 0))],
        out_specs=pl.BlockSpec((br, n), lambda i: (i, 0)),
        out_shape=jax.ShapeDtypeStruct((m, n), jnp.float32),
    )(x, w, b.reshape(1, n))


def _pn_edge_kernel(hxg_ref, ps_ref, pd_ref, w1b_ref, b1_ref, w2_ref, b2_ref,
                    o_ref):
    dpos = ps_ref[...] - pd_ref[...]
    h1 = hxg_ref[...] + jnp.dot(dpos, w1b_ref[...],
                                preferred_element_type=jnp.float32)
    h1 = jnp.maximum(h1 + b1_ref[...], 0.0)
    h2 = jnp.dot(h1, w2_ref[...], preferred_element_type=jnp.float32)
    o_ref[...] = jnp.maximum(h2 + b2_ref[...], 0.0)


def _pn_edge(hxg, ps, pd, w1b, b1, w2, b2):
    m = hxg.shape[0]
    return pl.pallas_call(
        _pn_edge_kernel,
        grid=(m // _BE,),
        in_specs=[pl.BlockSpec((_BE, _D), lambda i: (i, 0)),
                  pl.BlockSpec((_BE, 128), lambda i: (i, 0)),
                  pl.BlockSpec((_BE, 128), lambda i: (i, 0)),
                  pl.BlockSpec((128, _D), lambda i: (0, 0)),
                  pl.BlockSpec((1, _D), lambda i: (0, 0)),
                  pl.BlockSpec((_D, _D), lambda i: (0, 0)),
                  pl.BlockSpec((1, _D), lambda i: (0, 0))],
        out_specs=pl.BlockSpec((_BE, _D), lambda i: (i, 0)),
        out_shape=jax.ShapeDtypeStruct((m, _D), jnp.float32),
    )(hxg, ps, pd, w1b, b1.reshape(1, _D), w2, b2.reshape(1, _D))


def _pn_g_kernel(a_ref, w1_ref, b1_ref, w2_ref, b2_ref, o_ref):
    g = jnp.dot(a_ref[...], w1_ref[...],
                preferred_element_type=jnp.float32) + b1_ref[...]
    g = jnp.dot(g, w2_ref[...], preferred_element_type=jnp.float32) + b2_ref[...]
    o_ref[...] = jnp.maximum(g, 0.0)


def _pn_g3_kernel(a_ref, w1_ref, b1_ref, w2_ref, b2_ref, w3_ref, b3_ref,
                  o_ref):
    g = jnp.dot(a_ref[...], w1_ref[...],
                preferred_element_type=jnp.float32) + b1_ref[...]
    g = jnp.dot(g, w2_ref[...], preferred_element_type=jnp.float32) + b2_ref[...]
    g = jnp.maximum(g, 0.0)
    g = jnp.dot(g, w3_ref[...], preferred_element_type=jnp.float32) + b3_ref[...]
    o_ref[...] = jnp.maximum(g, 0.0)


def _pn_g(agg, p, last):
    m = agg.shape[0]
    wspec = pl.BlockSpec((_D, _D), lambda i: (0, 0))
    bspec = pl.BlockSpec((1, _D), lambda i: (0, 0))
    args = [agg, p["gW1"], p["gb1"].reshape(1, _D),
            p["gW2"], p["gb2"].reshape(1, _D)]
    specs = [pl.BlockSpec((_BR, _D), lambda i: (i, 0)), wspec, bspec, wspec,
             bspec]
    body = _pn_g_kernel
    if last:
        args += [p["gW3"], p["gb3"].reshape(1, _D)]
        specs += [wspec, bspec]
        body = _pn_g3_kernel
    return pl.pallas_call(
        body,
        grid=(m // _BR,),
        in_specs=specs,
        out_specs=pl.BlockSpec((_BR, _D), lambda i: (i, 0)),
        out_shape=jax.ShapeDtypeStruct((m, _D), jnp.float32),
    )(*args)


def _ew_kernel(x_ref, b_ref, o_ref, *, act):
    y = x_ref[...] + b_ref[...]
    if act:
        y = jnp.maximum(y, 0.0)
    o_ref[...] = y


def _ew(x, b, act):
    m, n = x.shape
    return pl.pallas_call(
        functools.partial(_ew_kernel, act=act),
        grid=(m // _BR,),
        in_specs=[pl.BlockSpec((_BR, n), lambda i: (i, 0)),
                  pl.BlockSpec((1, n), lambda i: (0, 0))],
        out_specs=pl.BlockSpec((_BR, n), lambda i: (i, 0)),
        out_shape=jax.ShapeDtypeStruct((m, n), jnp.float32),
    )(x, b.reshape(1, n))


def _mlp_kernel(ms_ref, w1_ref, b1_ref, w2_ref, b2_ref, w3_ref, b3_ref,
                o_ref):
    g = jnp.dot(ms_ref[...], w1_ref[...],
                preferred_element_type=jnp.float32) + b1_ref[...]
    g = jnp.dot(g, w2_ref[...], preferred_element_type=jnp.float32) + b2_ref[...]
    g = jnp.maximum(g, 0.0)
    g = jnp.dot(g, w3_ref[...], preferred_element_type=jnp.float32) + b3_ref[...]
    o_ref[...] = g


def _mlp(mode_stats, mp):
    msP = jnp.zeros((8, 8), jnp.float32).at[0, :2].set(mode_stats[0])
    w1P = jnp.zeros((8, 128), jnp.float32).at[:2].set(mp["m1_W"])
    out = pl.pallas_call(
        _mlp_kernel,
        out_shape=jax.ShapeDtypeStruct((8, _D), jnp.float32),
    )(msP, w1P, mp["m1_b"].reshape(1, 128),
      mp["m2_W"], mp["m2_b"].reshape(1, 128),
      mp["m3_W"], mp["m3_b"].reshape(1, _D))
    return out[0:1]


_NW = 32


def _sc_gather(table, idx, cols, chunk=32):
    m = idx.shape[0]
    per_w = m // _NW
    n_iter = per_w // chunk
    mesh = plsc.VectorSubcoreMesh(core_axis_name="c", subcore_axis_name="s")

    @functools.partial(
        pl.kernel,
        out_type=jax.ShapeDtypeStruct((m, cols), jnp.float32),
        mesh=mesh,
        scratch_types=[pltpu.VMEM((chunk,), jnp.int32),
                       pltpu.VMEM((chunk, cols), jnp.float32),
                       pltpu.SemaphoreType.DMA],
    )
    def k(table_hbm, idx_hbm, out_hbm, idx_v, rows_v, sem):
        wid = lax.axis_index("s") * 2 + lax.axis_index("c")
        base = wid * per_w

        def body(i, carry):
            off = base + i * chunk
            pltpu.sync_copy(idx_hbm.at[pl.ds(off, chunk)], idx_v)
            pltpu.async_copy(table_hbm.at[idx_v], rows_v, sem).wait()
            pltpu.sync_copy(rows_v, out_hbm.at[pl.ds(off, chunk)])
            return carry

        lax.fori_loop(0, n_iter, body, 0)

    return k(table, idx)


def kernel(x, pos, mode_stats, params, edge_index):
    n = _N
    loops = jnp.arange(n, dtype=edge_index.dtype)
    pad = _EP - _E_TOTAL
    spread = jnp.arange(pad, dtype=edge_index.dtype) % n
    src = jnp.concatenate([edge_index[0], loops, spread])
    dst = jnp.concatenate([edge_index[1], loops,
                           jnp.full((pad,), n, edge_index.dtype)])
    dst_c = jnp.concatenate([edge_index[1], loops, spread])

    xp = jnp.pad(x, ((0, _NP - n), (0, 0)))
    posall = jnp.pad(pos.reshape(n, 6), ((0, 0), (0, 122)))
    ps = _sc_gather(posall, src, 128)
    pd = _sc_gather(posall, dst_c, 128)

    h = xp
    for li, pname in enumerate(("pn1", "pn2", "pn3")):
        p = params[pname]
        w1a = p["lW1"][:_D]
        w1bP = jnp.zeros((128, _D), jnp.float32).at[2 * li:2 * li + 2].set(
            p["lW1"][_D:])
        hx = _dense(h, w1a)
        hxg = _sc_gather(hx, src, _D)
        h2 = _pn_edge(hxg, ps, pd, w1bP, p["lb1"], p["lW2"], p["lb2"])
        agg = jax.ops.segment_max(h2, dst, num_segments=n + 1)[:n]
        agg = jnp.pad(agg, ((0, _NP - n), (0, 0)))
        h = _pn_g(agg, p, last=(pname == "pn3"))

    for gi, gname in enumerate(("gat1", "gat2")):
        p = params[gname]
        hh = _dense(h, p["W"])
        attW = (jnp.zeros((_D, 128), jnp.float32)
                .at[:, 0].set(p["att_src"]).at[:, 1].set(p["att_dst"]))
        asd = _dense(hh, attW)
        a_s, a_d = asd[:, 0], asd[:, 1]
        e = jax.nn.leaky_relu(jnp.take(a_s, src, mode="clip")
                              + jnp.take(a_d, dst_c, mode="clip"), 0.2)
        mx = jax.ops.segment_max(e, dst, num_segments=n + 1)
        ex = jnp.exp(e - jnp.take(mx, dst, mode="clip"))
        den = jax.ops.segment_sum(ex, dst, num_segments=n + 1)
        alpha = ex / jnp.take(den, dst, mode="clip")
        hhg = _sc_gather(hh, src, _D)
        out = jax.ops.segment_sum(hhg * alpha[:, None], dst,
                                  num_segments=n + 1)[:n]
        out = jnp.pad(out, ((0, _NP - n), (0, 0)))
        h = _ew(out, p["b"], act=(gi == 0))

    g = _mlp(mode_stats, params["mlp"])
    return (h[:n], g)


# speedup vs baseline: 1.2508x; 1.0603x over previous
import functools

import jax
import jax.numpy as jnp
from jax import lax
from jax.experimental import pallas as pl
from jax.experimental.pallas import tpu as pltpu
from jax.experimental.pallas import tpu_sc as plsc

_N = 10000
_D = 256
_NP = 10240
_BR = 512
_E_TOTAL = 170000
_EP = 172032
_BE = 512


def _dense_kernel(x_ref, w_ref, b_ref, o_ref, *, act):
    y = jnp.dot(x_ref[...], w_ref[...], preferred_element_type=jnp.float32)
    y = y + b_ref[...]
    if act:
        y = jnp.maximum(y, 0.0)
    o_ref[...] = y


def _dense(x, w, b=None, act=False, br=_BR):
    m, k = x.shape
    n = w.shape[1]
    if b is None:
        b = jnp.zeros((n,), jnp.float32)
    return pl.pallas_call(
        functools.partial(_dense_kernel, act=act),
        grid=(m // br,),
        in_specs=[pl.BlockSpec((br, k), lambda i: (i, 0)),
                  pl.BlockSpec((k, n), lambda i: (0, 0)),
                  pl.BlockSpec((1, n), lambda i: (0, 0))],
        out_specs=pl.BlockSpec((br, n), lambda i: (i, 0)),
        out_shape=jax.ShapeDtypeStruct((m, n), jnp.float32),
    )(x, w, b.reshape(1, n))


def _pn_edge_kernel(hxg_ref, ps_ref, pd_ref, w1b_ref, b1_ref, w2_ref, b2_ref,
                    o_ref):
    dpos = ps_ref[...] - pd_ref[...]
    h1 = hxg_ref[...] + jnp.dot(dpos, w1b_ref[...],
                                preferred_element_type=jnp.float32)
    h1 = jnp.maximum(h1 + b1_ref[...], 0.0)
    h2 = jnp.dot(h1, w2_ref[...], preferred_element_type=jnp.float32)
    o_ref[...] = jnp.maximum(h2 + b2_ref[...], 0.0)


def _pn_edge(hxg, ps, pd, w1b, b1, w2, b2):
    m = hxg.shape[0]
    return pl.pallas_call(
        _pn_edge_kernel,
        grid=(m // _BE,),
        in_specs=[pl.BlockSpec((_BE, _D), lambda i: (i, 0)),
                  pl.BlockSpec((_BE, 128), lambda i: (i, 0)),
                  pl.BlockSpec((_BE, 128), lambda i: (i, 0)),
                  pl.BlockSpec((128, _D), lambda i: (0, 0)),
                  pl.BlockSpec((1, _D), lambda i: (0, 0)),
                  pl.BlockSpec((_D, _D), lambda i: (0, 0)),
                  pl.BlockSpec((1, _D), lambda i: (0, 0))],
        out_specs=pl.BlockSpec((_BE, _D), lambda i: (i, 0)),
        out_shape=jax.ShapeDtypeStruct((m, _D), jnp.float32),
    )(hxg, ps, pd, w1b, b1.reshape(1, _D), w2, b2.reshape(1, _D))


def _pn_g_kernel(a_ref, w1_ref, b1_ref, w2_ref, b2_ref, o_ref):
    g = jnp.dot(a_ref[...], w1_ref[...],
                preferred_element_type=jnp.float32) + b1_ref[...]
    g = jnp.dot(g, w2_ref[...], preferred_element_type=jnp.float32) + b2_ref[...]
    o_ref[...] = jnp.maximum(g, 0.0)


def _pn_g3_kernel(a_ref, w1_ref, b1_ref, w2_ref, b2_ref, w3_ref, b3_ref,
                  o_ref):
    g = jnp.dot(a_ref[...], w1_ref[...],
                preferred_element_type=jnp.float32) + b1_ref[...]
    g = jnp.dot(g, w2_ref[...], preferred_element_type=jnp.float32) + b2_ref[...]
    g = jnp.maximum(g, 0.0)
    g = jnp.dot(g, w3_ref[...], preferred_element_type=jnp.float32) + b3_ref[...]
    o_ref[...] = jnp.maximum(g, 0.0)


def _pn_g(agg, p, last):
    m = agg.shape[0]
    wspec = pl.BlockSpec((_D, _D), lambda i: (0, 0))
    bspec = pl.BlockSpec((1, _D), lambda i: (0, 0))
    args = [agg, p["gW1"], p["gb1"].reshape(1, _D),
            p["gW2"], p["gb2"].reshape(1, _D)]
    specs = [pl.BlockSpec((_BR, _D), lambda i: (i, 0)), wspec, bspec, wspec,
             bspec]
    body = _pn_g_kernel
    if last:
        args += [p["gW3"], p["gb3"].reshape(1, _D)]
        specs += [wspec, bspec]
        body = _pn_g3_kernel
    return pl.pallas_call(
        body,
        grid=(m // _BR,),
        in_specs=specs,
        out_specs=pl.BlockSpec((_BR, _D), lambda i: (i, 0)),
        out_shape=jax.ShapeDtypeStruct((m, _D), jnp.float32),
    )(*args)


def _ew_kernel(x_ref, b_ref, o_ref, *, act):
    y = x_ref[...] + b_ref[...]
    if act:
        y = jnp.maximum(y, 0.0)
    o_ref[...] = y


def _ew(x, b, act):
    m, n = x.shape
    return pl.pallas_call(
        functools.partial(_ew_kernel, act=act),
        grid=(m // _BR,),
        in_specs=[pl.BlockSpec((_BR, n), lambda i: (i, 0)),
                  pl.BlockSpec((1, n), lambda i: (0, 0))],
        out_specs=pl.BlockSpec((_BR, n), lambda i: (i, 0)),
        out_shape=jax.ShapeDtypeStruct((m, n), jnp.float32),
    )(x, b.reshape(1, n))


def _mlp_kernel(ms_ref, w1_ref, b1_ref, w2_ref, b2_ref, w3_ref, b3_ref,
                o_ref):
    g = jnp.dot(ms_ref[...], w1_ref[...],
                preferred_element_type=jnp.float32) + b1_ref[...]
    g = jnp.dot(g, w2_ref[...], preferred_element_type=jnp.float32) + b2_ref[...]
    g = jnp.maximum(g, 0.0)
    g = jnp.dot(g, w3_ref[...], preferred_element_type=jnp.float32) + b3_ref[...]
    o_ref[...] = g


def _mlp(mode_stats, mp):
    msP = jnp.zeros((8, 8), jnp.float32).at[0, :2].set(mode_stats[0])
    w1P = jnp.zeros((8, 128), jnp.float32).at[:2].set(mp["m1_W"])
    out = pl.pallas_call(
        _mlp_kernel,
        out_shape=jax.ShapeDtypeStruct((8, _D), jnp.float32),
    )(msP, w1P, mp["m1_b"].reshape(1, 128),
      mp["m2_W"], mp["m2_b"].reshape(1, 128),
      mp["m3_W"], mp["m3_b"].reshape(1, _D))
    return out[0:1]


_NW = 32


def _sc_gather(table, idx, cols, chunk=128, nbuf=2):
    m = idx.shape[0]
    per_w = m // _NW
    steps = per_w // (chunk * nbuf)
    mesh = plsc.VectorSubcoreMesh(core_axis_name="c", subcore_axis_name="s")

    @functools.partial(
        pl.kernel,
        out_type=jax.ShapeDtypeStruct((m, cols), jnp.float32),
        mesh=mesh,
        scratch_types=[pltpu.VMEM((per_w,), jnp.int32),
                       pltpu.VMEM((chunk, cols), jnp.float32),
                       pltpu.VMEM((chunk, cols), jnp.float32),
                       pltpu.SemaphoreType.DMA,
                       pltpu.SemaphoreType.DMA],
    )
    def k(table_hbm, idx_hbm, out_hbm, idx_v, r0, r1, s0, s1):
        wid = lax.axis_index("s") * 2 + lax.axis_index("c")
        base = wid * per_w
        pltpu.sync_copy(idx_hbm.at[pl.ds(base, per_w)], idx_v)
        rows = (r0, r1)
        sems = (s0, s1)

        def body(g, carry):
            offs = [pl.multiple_of(g * (chunk * nbuf) + b * chunk, chunk)
                    for b in range(nbuf)]
            cps = [pltpu.async_copy(
                       table_hbm.at[idx_v.at[pl.ds(offs[b], chunk)]],
                       rows[b], sems[b])
                   for b in range(nbuf)]
            for b in range(nbuf):
                cps[b].wait()
                pltpu.sync_copy(rows[b], out_hbm.at[pl.ds(base + offs[b], chunk)])
            return carry

        lax.fori_loop(0, steps, body, 0)

    return k(table, idx)


def kernel(x, pos, mode_stats, params, edge_index):
    n = _N
    loops = jnp.arange(n, dtype=edge_index.dtype)
    pad = _EP - _E_TOTAL
    spread = jnp.arange(pad, dtype=edge_index.dtype) % n
    src = jnp.concatenate([edge_index[0], loops, spread])
    dst = jnp.concatenate([edge_index[1], loops,
                           jnp.full((pad,), n, edge_index.dtype)])
    dst_c = jnp.concatenate([edge_index[1], loops, spread])

    xp = jnp.pad(x, ((0, _NP - n), (0, 0)))
    posall = jnp.pad(pos.reshape(n, 6), ((0, 0), (0, 122)))
    ps = _sc_gather(posall, src, 128)
    pd = _sc_gather(posall, dst_c, 128)

    h = xp
    for li, pname in enumerate(("pn1", "pn2", "pn3")):
        p = params[pname]
        w1a = p["lW1"][:_D]
        w1bP = jnp.zeros((128, _D), jnp.float32).at[2 * li:2 * li + 2].set(
            p["lW1"][_D:])
        hx = _dense(h, w1a)
        hxg = _sc_gather(hx, src, _D)
        h2 = _pn_edge(hxg, ps, pd, w1bP, p["lb1"], p["lW2"], p["lb2"])
        agg = jax.ops.segment_max(h2, dst, num_segments=n + 1)[:n]
        agg = jnp.pad(agg, ((0, _NP - n), (0, 0)))
        h = _pn_g(agg, p, last=(pname == "pn3"))

    for gi, gname in enumerate(("gat1", "gat2")):
        p = params[gname]
        hh = _dense(h, p["W"])
        attW = (jnp.zeros((_D, 128), jnp.float32)
                .at[:, 0].set(p["att_src"]).at[:, 1].set(p["att_dst"]))
        asd = _dense(hh, attW)
        a_s, a_d = asd[:, 0], asd[:, 1]
        e = jax.nn.leaky_relu(jnp.take(a_s, src, mode="clip")
                              + jnp.take(a_d, dst_c, mode="clip"), 0.2)
        mx = jax.ops.segment_max(e, dst, num_segments=n + 1)
        ex = jnp.exp(e - jnp.take(mx, dst, mode="clip"))
        den = jax.ops.segment_sum(ex, dst, num_segments=n + 1)
        alpha = ex / jnp.take(den, dst, mode="clip")
        hhg = _sc_gather(hh, src, _D)
        out = jax.ops.segment_sum(hhg * alpha[:, None], dst,
                                  num_segments=n + 1)[:n]
        out = jnp.pad(out, ((0, _NP - n), (0, 0)))
        h = _ew(out, p["b"], act=(gi == 0))

    g = _mlp(mode_stats, params["mlp"])
    return (h[:n], g)
